# baseline (speedup 1.0000x reference)
import jax
import jax.numpy as jnp
from jax import lax
from jax.experimental import pallas as pl
from jax.experimental.pallas import tpu as pltpu
from jax.experimental.pallas import tpu_sc as plsc

_NC = 2
_NS = 16
_NW = _NC * _NS
_L = 16

_B, _S, _H = 4, 8192, 1024
_S_SC = 3072
_RW = _S_SC // _NW
_CR = 16
_NCHUNK = _RW // _CR
_ITEMS = _NCHUNK * _B
_NXBUF = 4
_NPBUF = 2

_BS_TC = 1024
_OFF_TC = _S_SC // _BS_TC


def _sc_body(x_hbm, pe_hbm, out_hbm, scratch):
    pe_bufs = scratch["pe"]
    x_bufs = scratch["x"]
    pe_sems = scratch["pe_sem"]
    in_sems = scratch["in_sem"]
    out_sems = scratch["out_sem"]

    cid = lax.axis_index("c")
    sid = lax.axis_index("s")
    wid = sid * _NC + cid
    base = wid * _RW

    def rows(k):
        return pl.ds(base + (k // _B) * _CR, _CR)

    def pe_rows(ci):
        return pl.ds(base + ci * _CR, _CR)

    descs_in = [None] * _NXBUF
    descs_out = [None] * _NXBUF
    descs_pe = [None] * _NPBUF

    descs_pe[0] = pltpu.async_copy(
        pe_hbm.at[pe_rows(0)], pe_bufs[0], pe_sems[0])
    for j in range(_NXBUF - 1):
        descs_in[j] = pltpu.async_copy(
            x_hbm.at[j % _B, rows(j)], x_bufs[j], in_sems[j])

    for k in range(_ITEMS):
        ci, b = divmod(k, _B)
        buf = k % _NXBUF
        if b == 0:
            descs_pe[ci % _NPBUF].wait()
            descs_pe[ci % _NPBUF] = None
        kn = k + _NXBUF - 1
        if kn < _ITEMS:
            nbuf = kn % _NXBUF
            if descs_out[nbuf] is not None:
                descs_out[nbuf].wait()
                descs_out[nbuf] = None
            descs_in[nbuf] = pltpu.async_copy(
                x_hbm.at[kn % _B, rows(kn)], x_bufs[nbuf], in_sems[nbuf])
        if b == 0 and ci + 1 < _NCHUNK:
            nci = ci + 1
            descs_pe[nci % _NPBUF] = pltpu.async_copy(
                pe_hbm.at[pe_rows(nci)], pe_bufs[nci % _NPBUF],
                pe_sems[nci % _NPBUF])
        descs_in[buf].wait()
        descs_in[buf] = None

        x_v = x_bufs[buf]
        pe_v = pe_bufs[ci % _NPBUF]

        @plsc.parallel_loop(0, _CR * _H // _L, unroll=16)
        def _(i):
            r = lax.shift_right_logical(i, 6)
            c = pl.multiple_of(
                lax.shift_left(lax.bitwise_and(i, 63), 4), _L)
            plsc.addupdate(x_v.at[r, pl.ds(c, _L)], pe_v[r, pl.ds(c, _L)])

        descs_out[buf] = pltpu.async_copy(
            x_v, out_hbm.at[b, rows(k)], out_sems[buf])

    for buf in range(_NXBUF):
        if descs_out[buf] is not None:
            descs_out[buf].wait()


def _sc_slab(x, pe):
    mesh = plsc.VectorSubcoreMesh(core_axis_name="c", subcore_axis_name="s")
    return pl.kernel(
        _sc_body,
        out_type=jax.ShapeDtypeStruct((_B, _S_SC, _H), jnp.float32),
        mesh=mesh,
        cost_estimate=pl.CostEstimate(
            flops=_B * _S_SC * _H,
            bytes_accessed=(2 * _B + 1) * _S_SC * _H * 4,
            transcendentals=0,
        ),
        scratch_types=[{
            "pe": [pltpu.VMEM((_CR, _H), jnp.float32)] * _NPBUF,
            "x": [pltpu.VMEM((_CR, _H), jnp.float32)] * _NXBUF,
            "pe_sem": [pltpu.SemaphoreType.DMA] * _NPBUF,
            "in_sem": [pltpu.SemaphoreType.DMA] * _NXBUF,
            "out_sem": [pltpu.SemaphoreType.DMA] * _NXBUF,
        }],
    )(x, pe)


def _tc_body(x_ref, pe_ref, o_ref):
    o_ref[...] = x_ref[...] + pe_ref[...]


def _tc_slab(x, pe):
    B, S, H = x.shape
    grid = ((S - _S_SC) // _BS_TC, B)
    return pl.pallas_call(
        _tc_body,
        grid=grid,
        in_specs=[
            pl.BlockSpec((1, _BS_TC, H), lambda s, b: (b, s + _OFF_TC, 0)),
            pl.BlockSpec((_BS_TC, H), lambda s, b: (s + _OFF_TC, 0)),
        ],
        out_specs=pl.BlockSpec((1, _BS_TC, H), lambda s, b: (b, s + _OFF_TC, 0)),
        out_shape=jax.ShapeDtypeStruct((B, S, H), x.dtype),
        cost_estimate=pl.CostEstimate(
            flops=B * (S - _S_SC) * H,
            bytes_accessed=(2 * B + 1) * (S - _S_SC) * H * 4,
            transcendentals=0,
        ),
    )(x, pe)


def kernel(x, pe):
    sc_out = _sc_slab(x, pe)
    tc_out = _tc_slab(x, pe)
    return lax.dynamic_update_slice(tc_out, sc_out, (0, 0, 0))


# speedup vs baseline: 1.0950x; 1.0950x over previous
import jax
import jax.numpy as jnp
from jax import lax
from jax.experimental import pallas as pl
from jax.experimental.pallas import tpu as pltpu
from jax.experimental.pallas import tpu_sc as plsc

_NC = 2
_NS = 16
_NW = _NC * _NS
_L = 16

_B, _S, _H = 4, 8192, 1024
_RW = _S // _NW
_CR = 8
_NCHUNK = _RW // _CR
_NXBUF = 2 * _B
_NPBUF = 2


def _sc_body(x_hbm, pe_hbm, out_hbm, scratch):
    pe_bufs = scratch["pe"]
    x_bufs = scratch["x"]
    pe_sems = scratch["pe_sem"]
    in_sems = scratch["in_sem"]
    out_sems = scratch["out_sem"]

    cid = lax.axis_index("c")
    sid = lax.axis_index("s")
    wid = sid * _NC + cid
    base = wid * _RW

    def rows(ci):
        return pl.ds(base + ci * _CR, _CR)

    def bufs(ci):
        g = (ci % 2) * _B
        return [g + b for b in range(_B)]

    descs_in = [None] * _NXBUF
    descs_out = [None] * _NXBUF
    descs_pe = [None] * _NPBUF

    descs_pe[0] = pltpu.async_copy(pe_hbm.at[rows(0)], pe_bufs[0], pe_sems[0])
    descs_pe[1] = pltpu.async_copy(pe_hbm.at[rows(1)], pe_bufs[1], pe_sems[1])
    for b, j in enumerate(bufs(0)):
        descs_in[j] = pltpu.async_copy(
            x_hbm.at[b, rows(0)], x_bufs[j], in_sems[j])

    for ci in range(_NCHUNK):
        cur = bufs(ci)
        if ci + 1 < _NCHUNK:
            for b, j in enumerate(bufs(ci + 1)):
                if descs_out[j] is not None:
                    descs_out[j].wait()
                    descs_out[j] = None
                descs_in[j] = pltpu.async_copy(
                    x_hbm.at[b, rows(ci + 1)], x_bufs[j], in_sems[j])
        descs_pe[ci % _NPBUF].wait()
        descs_pe[ci % _NPBUF] = None
        if ci + 2 < _NCHUNK:
            descs_pe[ci % _NPBUF] = pltpu.async_copy(
                pe_hbm.at[rows(ci + 2)], pe_bufs[ci % _NPBUF],
                pe_sems[ci % _NPBUF])
        for j in cur:
            descs_in[j].wait()
            descs_in[j] = None

        pe_v = pe_bufs[ci % _NPBUF]
        xb = [x_bufs[j] for j in cur]

        @plsc.parallel_loop(0, _CR * _H // _L, unroll=8)
        def _(i):
            r = lax.shift_right_logical(i, 6)
            c = pl.multiple_of(
                lax.shift_left(lax.bitwise_and(i, 63), 4), _L)
            v = pe_v[r, pl.ds(c, _L)]
            for xv in xb:
                plsc.addupdate(xv.at[r, pl.ds(c, _L)], v)

        for b, j in enumerate(cur):
            descs_out[j] = pltpu.async_copy(
                x_bufs[j], out_hbm.at[b, rows(ci)], out_sems[j])

    for j in range(_NXBUF):
        if descs_out[j] is not None:
            descs_out[j].wait()


def kernel(x, pe):
    B, S, H = x.shape
    mesh = plsc.VectorSubcoreMesh(core_axis_name="c", subcore_axis_name="s")
    return pl.kernel(
        _sc_body,
        out_type=jax.ShapeDtypeStruct((B, S, H), jnp.float32),
        mesh=mesh,
        scratch_types=[{
            "pe": [pltpu.VMEM((_CR, _H), jnp.float32)] * _NPBUF,
            "x": [pltpu.VMEM((_CR, _H), jnp.float32)] * _NXBUF,
            "pe_sem": [pltpu.SemaphoreType.DMA] * _NPBUF,
            "in_sem": [pltpu.SemaphoreType.DMA] * _NXBUF,
            "out_sem": [pltpu.SemaphoreType.DMA] * _NXBUF,
        }],
    )(x, pe)


# speedup vs baseline: 1.1003x; 1.0048x over previous
import jax
import jax.numpy as jnp
from jax import lax
from jax.experimental import pallas as pl
from jax.experimental.pallas import tpu as pltpu
from jax.experimental.pallas import tpu_sc as plsc

_NC = 2
_NS = 16
_NW = _NC * _NS
_L = 16

_B, _S, _H = 4, 8192, 1024
_RW = _S // _NW
_CR = 8
_NCHUNK = _RW // _CR
_NXBUF = 2 * _B
_NPBUF = 2


def _sc_body(x_hbm, pe_hbm, out_hbm, scratch):
    pe_bufs = scratch["pe"]
    x_bufs = scratch["x"]
    pe_sems = scratch["pe_sem"]
    in_sems = scratch["in_sem"]
    out_sems = scratch["out_sem"]

    cid = lax.axis_index("c")
    sid = lax.axis_index("s")
    wid = sid * _NC + cid
    base = wid * _RW

    def rows(ci):
        return pl.ds(base + ci * _CR, _CR)

    def bufs(ci):
        g = (ci % 2) * _B
        return [g + b for b in range(_B)]

    descs_in = [None] * _NXBUF
    descs_out = [None] * _NXBUF
    descs_pe = [None] * _NPBUF

    descs_pe[0] = pltpu.async_copy(pe_hbm.at[rows(0)], pe_bufs[0], pe_sems[0])
    descs_pe[1] = pltpu.async_copy(pe_hbm.at[rows(1)], pe_bufs[1], pe_sems[1])
    for b, j in enumerate(bufs(0)):
        descs_in[j] = pltpu.async_copy(
            x_hbm.at[b, rows(0)], x_bufs[j], in_sems[j])

    for ci in range(_NCHUNK):
        cur = bufs(ci)
        if ci + 1 < _NCHUNK:
            for b, j in enumerate(bufs(ci + 1)):
                if descs_out[j] is not None:
                    descs_out[j].wait()
                    descs_out[j] = None
                descs_in[j] = pltpu.async_copy(
                    x_hbm.at[b, rows(ci + 1)], x_bufs[j], in_sems[j])
        descs_pe[ci % _NPBUF].wait()
        descs_pe[ci % _NPBUF] = None
        for j in cur:
            descs_in[j].wait()
            descs_in[j] = None

        pe_v = pe_bufs[ci % _NPBUF]
        xb = [x_bufs[j] for j in cur]

        @plsc.parallel_loop(0, _CR * _H // _L, unroll=8)
        def _(i):
            r = lax.shift_right_logical(i, 6)
            c = pl.multiple_of(
                lax.shift_left(lax.bitwise_and(i, 63), 4), _L)
            v = pe_v[r, pl.ds(c, _L)]
            for xv in xb:
                plsc.addupdate(xv.at[r, pl.ds(c, _L)], v)

        if ci + 2 < _NCHUNK:
            descs_pe[ci % _NPBUF] = pltpu.async_copy(
                pe_hbm.at[rows(ci + 2)], pe_bufs[ci % _NPBUF],
                pe_sems[ci % _NPBUF])

        for b, j in enumerate(cur):
            descs_out[j] = pltpu.async_copy(
                x_bufs[j], out_hbm.at[b, rows(ci)], out_sems[j])

    for j in range(_NXBUF):
        if descs_out[j] is not None:
            descs_out[j].wait()


def kernel(x, pe):
    B, S, H = x.shape
    mesh = plsc.VectorSubcoreMesh(core_axis_name="c", subcore_axis_name="s")
    return pl.kernel(
        _sc_body,
        out_type=jax.ShapeDtypeStruct((B, S, H), jnp.float32),
        mesh=mesh,
        scratch_types=[{
            "pe": [pltpu.VMEM((_CR, _H), jnp.float32)] * _NPBUF,
            "x": [pltpu.VMEM((_CR, _H), jnp.float32)] * _NXBUF,
            "pe_sem": [pltpu.SemaphoreType.DMA] * _NPBUF,
            "in_sem": [pltpu.SemaphoreType.DMA] * _NXBUF,
            "out_sem": [pltpu.SemaphoreType.DMA] * _NXBUF,
        }],
    )(x, pe)
